# Initial kernel scaffold; baseline (speedup 1.0000x reference)
import jax
import jax.numpy as jnp
from jax.experimental import pallas as pl


def kernel(jumped, edge_index):
    raise NotImplementedError("write your pallas kernel here")


# speedup vs baseline: 3.1827x; 3.1827x over previous
import functools

import jax
import jax.numpy as jnp
from jax import lax
from jax.experimental import pallas as pl
from jax.experimental.pallas import tpu as pltpu
from jax.experimental.pallas import tpu_sc as plsc

N_NODES_P = 10240
CHUNK = 128
CHUNKS_PER_TILE = 80
TILES = 16
E_PAD = 2 * TILES * CHUNKS_PER_TILE * CHUNK // 2
D_HALF = 128
ROWS_PER_TILE = N_NODES_P // TILES


def _sc_kernel():
    mesh = plsc.VectorSubcoreMesh(core_axis_name="c", subcore_axis_name="s")

    @functools.partial(
        pl.kernel,
        mesh=mesh,
        out_type=jax.ShapeDtypeStruct((2, N_NODES_P, D_HALF), jnp.float32),
        scratch_types=[
            pltpu.VMEM((CHUNKS_PER_TILE, CHUNK), jnp.int32),
            pltpu.VMEM((CHUNKS_PER_TILE, CHUNK), jnp.int32),
            pltpu.VMEM((CHUNK, D_HALF), jnp.float32),
            pltpu.VMEM_SHARED((N_NODES_P, D_HALF), jnp.float32),
            pltpu.SemaphoreType.DMA,
        ],
    )
    def k(table_hbm, src_hbm, dst_hbm, zeros_hbm, out_hbm,
          src_v, dst_v, buf, acc, sem):
        c = lax.axis_index("c")
        s = lax.axis_index("s")

        pltpu.sync_copy(src_hbm.at[c, s], src_v)
        pltpu.sync_copy(dst_hbm.at[s], dst_v)

        pltpu.sync_copy(zeros_hbm, buf)
        for r in range(ROWS_PER_TILE // CHUNK):
            pltpu.sync_copy(buf, acc.at[pl.ds(s * ROWS_PER_TILE + r * CHUNK, CHUNK)])
        plsc.subcore_barrier()

        def body(j, _):
            pltpu.async_copy(table_hbm.at[src_v.at[j]], buf, sem).wait()
            pltpu.sync_copy(buf, acc.at[dst_v.at[j]], add=True)
            return 0

        lax.fori_loop(0, CHUNKS_PER_TILE, body, 0)
        plsc.subcore_barrier()

        pltpu.sync_copy(acc.at[pl.ds(s * ROWS_PER_TILE, ROWS_PER_TILE)],
                        out_hbm.at[c, pl.ds(s * ROWS_PER_TILE, ROWS_PER_TILE)])

    return k


_k = _sc_kernel()


@jax.jit
def kernel(jumped, edge_index):
    n_nodes, d = jumped.shape
    n_edges = edge_index.shape[1]

    table = jnp.concatenate([jumped[:, :D_HALF], jumped[:, D_HALF:]], axis=0)

    pad = E_PAD - n_edges
    src = jnp.concatenate([edge_index[0], jnp.zeros((pad,), jnp.int32)])
    dst = jnp.concatenate([edge_index[1], jnp.full((pad,), n_nodes, jnp.int32)])
    src2 = jnp.stack([src, src + n_nodes]).reshape(2, TILES, CHUNKS_PER_TILE, CHUNK)
    dst3 = dst.reshape(TILES, CHUNKS_PER_TILE, CHUNK)
    zeros = jnp.zeros((CHUNK, D_HALF), jnp.float32)

    out = _k(table, src2, dst3, zeros)
    return jnp.concatenate([out[0, :n_nodes], out[1, :n_nodes]], axis=1)


# speedup vs baseline: 3.7661x; 1.1833x over previous
import functools

import jax
import jax.numpy as jnp
from jax import lax
from jax.experimental import pallas as pl
from jax.experimental.pallas import tpu as pltpu
from jax.experimental.pallas import tpu_sc as plsc

N_NODES_P = 10240
CHUNK = 128
CHUNKS_PER_TILE = 80
TILES = 16
E_PAD = TILES * CHUNKS_PER_TILE * CHUNK
D_HALF = 128
ROWS_PER_TILE = N_NODES_P // TILES
NSLOT = 4


def _sc_kernel():
    mesh = plsc.VectorSubcoreMesh(core_axis_name="c", subcore_axis_name="s")

    @functools.partial(
        pl.kernel,
        mesh=mesh,
        out_type=jax.ShapeDtypeStruct((2, N_NODES_P, D_HALF), jnp.float32),
        scratch_types=[
            pltpu.VMEM((CHUNK, D_HALF), jnp.float32),
            pltpu.VMEM((CHUNK, D_HALF), jnp.float32),
            pltpu.VMEM((NSLOT, 2, CHUNK), jnp.int32),
            pltpu.VMEM_SHARED((N_NODES_P, D_HALF), jnp.float32),
            pltpu.SemaphoreType.DMA,
            pltpu.SemaphoreType.DMA,
            pltpu.SemaphoreType.DMA,
            pltpu.SemaphoreType.DMA,
            pltpu.SemaphoreType.DMA,
            pltpu.SemaphoreType.DMA,
        ],
    )
    def k(table_hbm, eidx_hbm, zeros_hbm, out_hbm,
          buf_a, buf_b, islots, acc,
          gsem_a, gsem_b, isem0, isem1, isem2, isem3):
        c = lax.axis_index("c")
        s = lax.axis_index("s")
        gbufs = (buf_a, buf_b)
        gsems = (gsem_a, gsem_b)
        isems = (isem0, isem1, isem2, isem3)

        pltpu.sync_copy(zeros_hbm, buf_a)
        for r in range(ROWS_PER_TILE // CHUNK):
            pltpu.sync_copy(buf_a, acc.at[pl.ds(s * ROWS_PER_TILE + r * CHUNK, CHUNK)])
        plsc.subcore_barrier()

        def issue_idx(j, sl):
            pltpu.async_copy(eidx_hbm.at[c, s, j], islots.at[sl], isems[sl])

        def wait_idx(j, sl):
            pltpu.make_async_copy(eidx_hbm.at[c, s, j], islots.at[sl],
                                  isems[sl]).wait()

        def issue_gather(sl, b):
            pltpu.async_copy(table_hbm.at[islots.at[sl, 0]], gbufs[b], gsems[b])

        def wait_gather(sl, b):
            pltpu.make_async_copy(table_hbm.at[islots.at[sl, 0]], gbufs[b],
                                  gsems[b]).wait()

        for t in range(NSLOT):
            issue_idx(t, t)
        wait_idx(0, 0)
        issue_gather(0, 0)
        wait_idx(1, 1)
        issue_gather(1, 1)

        @pl.loop(0, CHUNKS_PER_TILE, step=NSLOT)
        def _(j):
            for b in range(NSLOT):
                jj = j + b
                g = b % 2
                wait_gather(b, g)
                pltpu.sync_copy(gbufs[g], acc.at[islots.at[b, 1]], add=True)

                @pl.when(jj + NSLOT < CHUNKS_PER_TILE)
                def _():
                    issue_idx(jj + NSLOT, b)

                @pl.when(jj + 2 < CHUNKS_PER_TILE)
                def _():
                    wait_idx(jj + 2, (b + 2) % NSLOT)
                    issue_gather((b + 2) % NSLOT, g)

        plsc.subcore_barrier()

        pltpu.sync_copy(acc.at[pl.ds(s * ROWS_PER_TILE, ROWS_PER_TILE)],
                        out_hbm.at[c, pl.ds(s * ROWS_PER_TILE, ROWS_PER_TILE)])

    return k


_k = _sc_kernel()


@jax.jit
def kernel(jumped, edge_index):
    n_nodes, d = jumped.shape
    n_edges = edge_index.shape[1]

    table = jnp.concatenate([jumped[:, :D_HALF], jumped[:, D_HALF:]], axis=0)

    pad = E_PAD - n_edges
    src = jnp.concatenate([edge_index[0], jnp.zeros((pad,), jnp.int32)])
    dst = jnp.concatenate([edge_index[1], jnp.full((pad,), n_nodes, jnp.int32)])
    src_r = src.reshape(TILES, CHUNKS_PER_TILE, CHUNK)
    dst_r = dst.reshape(TILES, CHUNKS_PER_TILE, CHUNK)
    eidx = jnp.stack([
        jnp.stack([src_r, dst_r], axis=2),
        jnp.stack([src_r + n_nodes, dst_r], axis=2),
    ])
    zeros = jnp.zeros((CHUNK, D_HALF), jnp.float32)

    out = _k(table, eidx, zeros)
    return jnp.concatenate([out[0, :n_nodes], out[1, :n_nodes]], axis=1)
